# baseline (speedup 1.0000x reference)
import jax
import jax.numpy as jnp
from jax import lax
from jax.experimental import pallas as pl
from jax.experimental.pallas import tpu as pltpu
from jax.experimental.pallas import tpu_sc as plsc

B, P, D = 16384, 200, 64
LANES = 16


_TAB_BLK = 8


def _table_body(pe_ref, cp_ref, out_ref):
    cp = cp_ref[...]
    pe = pe_ref[...]
    out_ref[...] = cp[None, :, :] + pe[:, None, :]


def _build_table(c_perm, pos_emb):
    return pl.pallas_call(
        _table_body,
        grid=(P // _TAB_BLK,),
        in_specs=[
            pl.BlockSpec((_TAB_BLK, D), lambda i: (i, 0)),
            pl.BlockSpec((P, D), lambda i: (0, 0)),
        ],
        out_specs=pl.BlockSpec((_TAB_BLK, P, D), lambda i: (i, 0, 0)),
        out_shape=jax.ShapeDtypeStruct((P, P, D), jnp.float32),
    )(pos_emb, c_perm).reshape(P * P, D)


_PX_BLK = 1024
GP = 64


def _perm_body(x_ref, out_ref):
    cc = pl.program_id(0)
    xv = x_ref[...].astype(jnp.float32)
    vv = lax.broadcasted_iota(jnp.int32, (P, NC * GP), 0)
    tt = lax.broadcasted_iota(jnp.int32, (P, NC * GP), 1)
    h = tt // GP
    j = tt - h * GP
    srcp = cc * (P // NC) + 2 * j + h
    valid = j < (P // NC) // 2
    perm = jnp.where((vv == srcp) & valid, 1.0, 0.0)
    y = jnp.dot(xv, perm, preferred_element_type=jnp.float32)
    off = jnp.where(valid, (2 * j + h) * P, 0)[:1, :]
    yi = y.astype(jnp.int32) + off
    out_ref[...] = yi


def _permute_idx(x):
    return pl.pallas_call(
        _perm_body,
        grid=(NC, B // _PX_BLK),
        in_specs=[pl.BlockSpec((_PX_BLK, P), lambda cc, k: (k, 0))],
        out_specs=pl.BlockSpec(
            (_PX_BLK * NC * GP // 128, 128),
            lambda cc, k: (cc * (B // _PX_BLK) + k, 0)),
        out_shape=jax.ShapeDtypeStruct((NC * B * NC * GP // 128, 128),
                                       jnp.int32),
    )(x)


NC = 2
NS = 16
HP = P // NC
TROWS = HP * P
STEPS_PER_BLK = 8
NR = 4
NCH = 4
BCH = B // NCH
B_PER_S = BCH // NS
NBLK = B_PER_S // STEPS_PER_BLK


def _sc_body(ch, x_hbm, tab_hbm, out_hbm, idx_v, rows_v, tab_sh,
             sem_i, sem_g, sem_o):
    c = lax.axis_index("c")
    s = lax.axis_index("s")
    cb0 = ch * BCH

    @pl.when(s == 0)
    def _stage_table():
        pltpu.sync_copy(tab_hbm.at[pl.ds(c * TROWS, TROWS)], tab_sh)

    plsc.subcore_barrier()

    def _idx_copy(k, pb):
        b0 = s * B_PER_S + k * STEPS_PER_BLK
        return pltpu.make_async_copy(
            x_hbm.at[c, pl.ds(b0, STEPS_PER_BLK)],
            idx_v.at[pb],
            sem_i,
        )

    def _wb_half(slot, bi, h):
        row2 = (bi - cb0) * (2 * GP) + c * GP
        return pltpu.make_async_copy(
            rows_v.at[slot, h],
            out_hbm.at[pl.ds(row2, GP), pl.ds(h * D, D)],
            sem_o.at[slot],
        )

    def _wb_start(slot, bi):
        _wb_half(slot, bi, 0).start()
        _wb_half(slot, bi, 1).start()

    def _wb_wait(slot, bi):
        _wb_half(slot, bi, 0).wait()
        _wb_half(slot, bi, 1).wait()

    def _g_copy(slot, pb, i, h):
        return pltpu.make_async_copy(
            tab_sh.at[idx_v.at[pb, i, h]],
            rows_v.at[slot, h],
            sem_g.at[slot],
        )

    _idx_copy(0, 0).start()

    def run_block(k, pb):
        bi0 = cb0 + s * B_PER_S + k * STEPS_PER_BLK

        _idx_copy(k, pb).wait()

        @pl.when(k + 1 < NBLK)
        def _prefetch():
            _idx_copy(k + 1, 1 - pb).start()

        for i in range(STEPS_PER_BLK):
            slot = i % NR
            if i < NR:
                @pl.when(k >= 1)
                def _reclaim():
                    _wb_wait(slot, bi0 - (NR - i))
            else:
                _wb_wait(slot, bi0 + (i - NR))
            _g_copy(slot, pb, i, 0).start()
            _g_copy(slot, pb, i, 1).start()
            if i >= 1:
                pslot = (i - 1) % NR
                _g_copy(pslot, pb, i - 1, 0).wait()
                _g_copy(pslot, pb, i - 1, 1).wait()
                _wb_start(pslot, bi0 + i - 1)
        last = STEPS_PER_BLK - 1
        lslot = last % NR
        _g_copy(lslot, pb, last, 0).wait()
        _g_copy(lslot, pb, last, 1).wait()
        _wb_start(lslot, bi0 + last)

    def outer(gg, carry):
        for parity in range(2):
            run_block(gg * 2 + parity, parity)
        return carry

    lax.fori_loop(0, NBLK // 2, outer, 0)

    bend = cb0 + s * B_PER_S + B_PER_S
    for slot in range(NR):
        _wb_wait(slot, bend - (NR - slot))


def _sc_gather(xb, table, ch):
    import functools
    mesh = plsc.VectorSubcoreMesh(core_axis_name="c", subcore_axis_name="s")
    run = pl.kernel(
        functools.partial(_sc_body, ch),
        out_type=jax.ShapeDtypeStruct((BCH * 2 * GP, 128), jnp.float32),
        mesh=mesh,
        compiler_params=pltpu.CompilerParams(use_tc_tiling_on_sc=False),
        scratch_types=[
            pltpu.VMEM((2, STEPS_PER_BLK, 2, GP), jnp.int32),
            pltpu.VMEM((NR, 2, GP, D), jnp.float32),
            pltpu.VMEM_SHARED((TROWS, D), jnp.float32),
            pltpu.SemaphoreType.DMA,
            pltpu.SemaphoreType.DMA((NR,)),
            pltpu.SemaphoreType.DMA((NR,)),
        ],
    )
    return run(xb, table)


_T_BB = 128


def _compact(v):
    w = v.reshape(_T_BB, 2 * GP, 128)
    real = jnp.concatenate([w[:, 0:HP // 2, :], w[:, GP:GP + HP // 2, :]],
                           axis=1)
    return real.reshape(_T_BB, P * D).T


def _tr_body(in_ref, out_ref):
    out_ref[...] = _compact(in_ref[...])


def _tr_body_alias(big_ref, in_ref, out_ref):
    out_ref[...] = _compact(in_ref[...])


def _transpose_first(g0):
    return pl.pallas_call(
        _tr_body,
        grid=(BCH // _T_BB,),
        in_specs=[pl.BlockSpec((_T_BB * 2 * GP, 128), lambda i: (i, 0))],
        out_specs=pl.BlockSpec((P * D, _T_BB), lambda i: (0, i)),
        out_shape=jax.ShapeDtypeStruct((P * D, B), jnp.float32),
    )(g0)


def _transpose_next(big, g, ch):
    nblk0 = ch * (BCH // _T_BB)
    return pl.pallas_call(
        _tr_body_alias,
        grid=(BCH // _T_BB,),
        in_specs=[
            pl.BlockSpec((8, 128), lambda i: (0, 0)),
            pl.BlockSpec((_T_BB * 2 * GP, 128), lambda i: (i, 0)),
        ],
        out_specs=pl.BlockSpec((P * D, _T_BB), lambda i: (0, nblk0 + i)),
        out_shape=jax.ShapeDtypeStruct((P * D, B), jnp.float32),
        input_output_aliases={0: 0},
    )(big, g)


def kernel(x, c_perm, pos_emb):
    table = _build_table(c_perm, pos_emb)
    xb = _permute_idx(x).reshape(NC, B, 2, GP)
    xbs = [xb[:, ch * BCH:(ch + 1) * BCH] for ch in range(NCH)]
    gs = [_sc_gather(xbs[ch], table, ch) for ch in range(NCH)]
    out_t = _transpose_first(gs[0])
    for ch in range(1, NCH):
        out_t = _transpose_next(out_t, gs[ch], ch)
    return jnp.transpose(out_t.reshape(P, D, B), (2, 0, 1))


# speedup vs baseline: 1.2502x; 1.2502x over previous
import jax
import jax.numpy as jnp
from jax import lax
from jax.experimental import pallas as pl
from jax.experimental.pallas import tpu as pltpu
from jax.experimental.pallas import tpu_sc as plsc

B, P, D = 16384, 200, 64
LANES = 16


_TAB_BLK = 8


def _table_body(pe_ref, cp_ref, out_ref):
    cp = cp_ref[...]
    pe = pe_ref[...]
    out_ref[...] = cp[None, :, :] + pe[:, None, :]


def _build_table(c_perm, pos_emb):
    return pl.pallas_call(
        _table_body,
        grid=(P // _TAB_BLK,),
        in_specs=[
            pl.BlockSpec((_TAB_BLK, D), lambda i: (i, 0)),
            pl.BlockSpec((P, D), lambda i: (0, 0)),
        ],
        out_specs=pl.BlockSpec((_TAB_BLK, P, D), lambda i: (i, 0, 0)),
        out_shape=jax.ShapeDtypeStruct((P, P, D), jnp.float32),
    )(pos_emb, c_perm).reshape(P * P, D)


_X_BLK = 1024


def _xoff_body(x_ref, out_ref):
    pos = jax.lax.broadcasted_iota(jnp.int32, (_X_BLK, P), 1)
    out_ref[...] = x_ref[...] + (pos % (P // 2)) * P


def _linearize(x):
    return pl.pallas_call(
        _xoff_body,
        grid=(B // _X_BLK,),
        in_specs=[pl.BlockSpec((_X_BLK, P), lambda i: (i, 0))],
        out_specs=pl.BlockSpec((_X_BLK, P), lambda i: (i, 0)),
        out_shape=jax.ShapeDtypeStruct((B, P), jnp.int32),
    )(x)


NC = 2
NS = 16
HP = P // NC
TROWS = HP * P
STEPS_PER_BLK = 8
NR = 4
NCH = 4
BCH = B // NCH
B_PER_S = BCH // NS
NBLK = B_PER_S // STEPS_PER_BLK


def _sc_body(ch, x_hbm, tab_hbm, out_hbm, idx_v, rows_v, tab_sh,
             sem_i, sem_g, sem_o):
    c = lax.axis_index("c")
    s = lax.axis_index("s")
    cb0 = ch * BCH

    @pl.when(s == 0)
    def _stage_table():
        pltpu.sync_copy(tab_hbm.at[pl.ds(c * TROWS, TROWS)], tab_sh)

    plsc.subcore_barrier()

    def _idx_copy(k, pb):
        b0 = s * B_PER_S + k * STEPS_PER_BLK
        return pltpu.make_async_copy(
            x_hbm.at[c, pl.ds(b0, STEPS_PER_BLK)],
            idx_v.at[pb],
            sem_i,
        )

    def _wb_half(slot, bi, h):
        row2 = (bi - cb0) * (P * D // 128) + c * (HP * D // 128)
        return pltpu.make_async_copy(
            rows_v.at[slot, h],
            out_hbm.at[pl.ds(row2, HP * D // 128), pl.ds(h * D, D)],
            sem_o.at[slot],
        )

    def _wb_start(slot, bi):
        _wb_half(slot, bi, 0).start()
        _wb_half(slot, bi, 1).start()

    def _wb_wait(slot, bi):
        _wb_half(slot, bi, 0).wait()
        _wb_half(slot, bi, 1).wait()

    def _g_copy(slot, pb, i, h):
        return pltpu.make_async_copy(
            tab_sh.at[idx_v.at[pb, i, h]],
            rows_v.at[slot, h],
            sem_g.at[slot],
        )

    _idx_copy(0, 0).start()

    def run_block(k, pb):
        bi0 = cb0 + s * B_PER_S + k * STEPS_PER_BLK

        _idx_copy(k, pb).wait()

        @pl.when(k + 1 < NBLK)
        def _prefetch():
            _idx_copy(k + 1, 1 - pb).start()

        for i in range(STEPS_PER_BLK):
            slot = i % NR
            if i < NR:
                @pl.when(k >= 1)
                def _reclaim():
                    _wb_wait(slot, bi0 - (NR - i))
            else:
                _wb_wait(slot, bi0 + (i - NR))
            _g_copy(slot, pb, i, 0).start()
            _g_copy(slot, pb, i, 1).start()
            if i >= 1:
                pslot = (i - 1) % NR
                _g_copy(pslot, pb, i - 1, 0).wait()
                _g_copy(pslot, pb, i - 1, 1).wait()
                _wb_start(pslot, bi0 + i - 1)
        last = STEPS_PER_BLK - 1
        lslot = last % NR
        _g_copy(lslot, pb, last, 0).wait()
        _g_copy(lslot, pb, last, 1).wait()
        _wb_start(lslot, bi0 + last)

    def outer(gg, carry):
        for parity in range(2):
            run_block(gg * 2 + parity, parity)
        return carry

    lax.fori_loop(0, NBLK // 2, outer, 0)

    bend = cb0 + s * B_PER_S + B_PER_S
    for slot in range(NR):
        _wb_wait(slot, bend - (NR - slot))


def _sc_gather(xb, table, ch):
    import functools
    mesh = plsc.VectorSubcoreMesh(core_axis_name="c", subcore_axis_name="s")
    run = pl.kernel(
        functools.partial(_sc_body, ch),
        out_type=jax.ShapeDtypeStruct((BCH * P * D // 128, 128), jnp.float32),
        mesh=mesh,
        compiler_params=pltpu.CompilerParams(use_tc_tiling_on_sc=False),
        scratch_types=[
            pltpu.VMEM((2, STEPS_PER_BLK, 2, HP // 2), jnp.int32),
            pltpu.VMEM((NR, 2, HP // 2, D), jnp.float32),
            pltpu.VMEM_SHARED((TROWS, D), jnp.float32),
            pltpu.SemaphoreType.DMA,
            pltpu.SemaphoreType.DMA((NR,)),
            pltpu.SemaphoreType.DMA((NR,)),
        ],
    )
    return run(xb, table)


_T_BB = 128


def _tr_body(in_ref, out_ref):
    v = in_ref[...]
    out_ref[...] = v.reshape(_T_BB, P * D).T


def _tr_body_alias(big_ref, in_ref, out_ref):
    v = in_ref[...]
    out_ref[...] = v.reshape(_T_BB, P * D).T


def _transpose_first(g0):
    return pl.pallas_call(
        _tr_body,
        grid=(BCH // _T_BB,),
        in_specs=[pl.BlockSpec((_T_BB * P * D // 128, 128), lambda i: (i, 0))],
        out_specs=pl.BlockSpec((P * D, _T_BB), lambda i: (0, i)),
        out_shape=jax.ShapeDtypeStruct((P * D, B), jnp.float32),
    )(g0)


def _transpose_next(big, g, ch):
    nblk0 = ch * (BCH // _T_BB)
    return pl.pallas_call(
        _tr_body_alias,
        grid=(BCH // _T_BB,),
        in_specs=[
            pl.BlockSpec((8, 128), lambda i: (0, 0)),
            pl.BlockSpec((_T_BB * P * D // 128, 128), lambda i: (i, 0)),
        ],
        out_specs=pl.BlockSpec((P * D, _T_BB), lambda i: (0, nblk0 + i)),
        out_shape=jax.ShapeDtypeStruct((P * D, B), jnp.float32),
        input_output_aliases={0: 0},
    )(big, g)


def kernel(x, c_perm, pos_emb):
    table = _build_table(c_perm, pos_emb)
    xoff = _linearize(x)
    xbs = [
        jnp.transpose(
            xoff[ch * BCH:(ch + 1) * BCH].reshape(BCH, NC, HP // 2, 2),
            (1, 0, 3, 2))
        for ch in range(NCH)
    ]
    gs = [_sc_gather(xbs[ch], table, ch) for ch in range(NCH)]
    out_t = _transpose_first(gs[0])
    for ch in range(1, NCH):
        out_t = _transpose_next(out_t, gs[ch], ch)
    return jnp.transpose(out_t.reshape(P, D, B), (2, 0, 1))
